# Initial kernel scaffold; baseline (speedup 1.0000x reference)
import jax
import jax.numpy as jnp
from jax.experimental import pallas as pl


def kernel(x, hyperedge_index, W1, b1, W2, b2, W3, b3, W4, b4, W5, b5, gamma, beta):
    raise NotImplementedError("write your pallas kernel here")


# speedup vs baseline: 8.2807x; 8.2807x over previous
import functools

import jax
import jax.numpy as jnp
from jax import lax
from jax.experimental import pallas as pl
from jax.experimental.pallas import tpu as pltpu
from jax.experimental.pallas import tpu_sc as plsc

N = 10000
E = 320000
D = 128
NPAD = 10240
NC = 2
NS = 16
NT = NC * NS
EPT = E // NT
C = 200
CHUNKS = EPT // C
STRIPE = NPAD // NS

_mesh = plsc.VectorSubcoreMesh(core_axis_name="c", subcore_axis_name="s")

_f32 = jnp.float32


@functools.partial(
    pl.kernel,
    mesh=_mesh,
    out_type=[
        jax.ShapeDtypeStruct((NC, NPAD, D), _f32),
        jax.ShapeDtypeStruct((NC, NPAD, D), _f32),
    ],
    scratch_types=[
        pltpu.VMEM((C,), jnp.int32),
        pltpu.VMEM((C, D), _f32),
        pltpu.VMEM_SHARED((NPAD, D), _f32),
    ],
)
def _sc_counts(nidx_hbm, eidx_hbm, ones_hbm, zeros_hbm,
               dcnt_hbm, bcnt_hbm, idx_v, ones_v, acc_sh):
    c = lax.axis_index("c")
    s = lax.axis_index("s")
    pltpu.sync_copy(ones_hbm, ones_v)
    base = (c * NS + s) * EPT

    for idx_hbm, out_hbm in ((nidx_hbm, dcnt_hbm), (eidx_hbm, bcnt_hbm)):
        pltpu.sync_copy(zeros_hbm.at[pl.ds(s * STRIPE, STRIPE)],
                        acc_sh.at[pl.ds(s * STRIPE, STRIPE)])
        plsc.subcore_barrier()

        @pl.loop(0, CHUNKS)
        def _(j):
            off = base + j * C
            pltpu.sync_copy(idx_hbm.at[pl.ds(off, C)], idx_v)
            pltpu.sync_copy(ones_v, acc_sh.at[idx_v], add=True)

        plsc.subcore_barrier()
        pltpu.sync_copy(acc_sh.at[pl.ds(s * STRIPE, STRIPE)],
                        out_hbm.at[c, pl.ds(s * STRIPE, STRIPE)])
        plsc.subcore_barrier()


@functools.partial(
    pl.kernel,
    mesh=_mesh,
    out_type=jax.ShapeDtypeStruct((NC, NPAD, D), _f32),
    scratch_types=[
        pltpu.VMEM((C,), jnp.int32),
        pltpu.VMEM((C,), jnp.int32),
        pltpu.VMEM((C, D), _f32),
        pltpu.VMEM_SHARED((NPAD, D), _f32),
        pltpu.SemaphoreType.DMA,
    ],
)
def _sc_stage(feat_hbm, gidx_hbm, sidx_hbm, zeros_hbm,
              out_hbm, gi_v, si_v, rows_v, acc_sh, sem):
    c = lax.axis_index("c")
    s = lax.axis_index("s")
    pltpu.sync_copy(zeros_hbm.at[pl.ds(s * STRIPE, STRIPE)],
                    acc_sh.at[pl.ds(s * STRIPE, STRIPE)])
    plsc.subcore_barrier()
    base = (c * NS + s) * EPT

    @pl.loop(0, CHUNKS)
    def _(j):
        off = base + j * C
        pltpu.sync_copy(gidx_hbm.at[pl.ds(off, C)], gi_v)
        pltpu.sync_copy(sidx_hbm.at[pl.ds(off, C)], si_v)
        pltpu.async_copy(feat_hbm.at[gi_v], rows_v, sem).wait()
        pltpu.sync_copy(rows_v, acc_sh.at[si_v], add=True)

    plsc.subcore_barrier()
    pltpu.sync_copy(acc_sh.at[pl.ds(s * STRIPE, STRIPE)],
                    out_hbm.at[c, pl.ds(s * STRIPE, STRIPE)])


_RB = 400
_GRID = N // _RB


def _inv_body(d_ref, b_ref, dinv_ref, binv_ref):
    ds_ = d_ref[0, :, 0:1] + d_ref[1, :, 0:1]
    bs_ = b_ref[0, :, 0:1] + b_ref[1, :, 0:1]
    dinv = jnp.where(ds_ > 0, 1.0 / ds_, 0.0)
    binv = jnp.where(bs_ > 0, 1.0 / bs_, 0.0)
    dinv_ref[...] = jnp.broadcast_to(dinv, (512, D))
    binv_ref[...] = jnp.broadcast_to(binv, (512, D))


def _tc_inv(dcnt, bcnt):
    return pl.pallas_call(
        _inv_body,
        grid=(NPAD // 512,),
        in_specs=[
            pl.BlockSpec((NC, 512, D), lambda i: (0, i, 0)),
            pl.BlockSpec((NC, 512, D), lambda i: (0, i, 0)),
        ],
        out_specs=[
            pl.BlockSpec((512, D), lambda i: (i, 0)),
            pl.BlockSpec((512, D), lambda i: (i, 0)),
        ],
        out_shape=[
            jax.ShapeDtypeStruct((NPAD, D), _f32),
            jax.ShapeDtypeStruct((NPAD, D), _f32),
        ],
    )(dcnt, bcnt)


def _mm_body(x_ref, w_ref, o_ref):
    o_ref[...] = jnp.dot(x_ref[...], w_ref[...],
                         preferred_element_type=_f32)


def _tc_mm(x, w):
    return pl.pallas_call(
        _mm_body,
        grid=(_GRID,),
        in_specs=[
            pl.BlockSpec((_RB, D), lambda i: (i, 0)),
            pl.BlockSpec((D, D), lambda i: (0, 0)),
        ],
        out_specs=pl.BlockSpec((_RB, D), lambda i: (i, 0)),
        out_shape=jax.ShapeDtypeStruct((N, D), _f32),
    )(x, w)


def _combine_body(p_ref, binv_ref, o_ref):
    o_ref[...] = binv_ref[...] * (p_ref[0] + p_ref[1])


def _tc_combine(part, binv_b):
    return pl.pallas_call(
        _combine_body,
        grid=(_GRID,),
        in_specs=[
            pl.BlockSpec((NC, _RB, D), lambda i: (0, i, 0)),
            pl.BlockSpec((_RB, D), lambda i: (i, 0)),
        ],
        out_specs=pl.BlockSpec((_RB, D), lambda i: (i, 0)),
        out_shape=jax.ShapeDtypeStruct((N, D), _f32),
    )(part, binv_b)


def _post_body(p_ref, dinv_ref, bias_ref, gamma_ref, beta_ref, w_ref, o_ref,
               *, use_ln):
    h = dinv_ref[...] * (p_ref[0] + p_ref[1]) + bias_ref[...]
    if use_ln:
        mu = jnp.mean(h, axis=-1, keepdims=True)
        var = jnp.mean((h - mu) ** 2, axis=-1, keepdims=True)
        h = (h - mu) / jnp.sqrt(var + 1e-5) * gamma_ref[...] + beta_ref[...]
    h = jnp.maximum(h, 0.0)
    o_ref[...] = jnp.dot(h, w_ref[...], preferred_element_type=_f32)


def _tc_post(part, dinv_b, bias, gamma, beta, w_next, use_ln):
    return pl.pallas_call(
        functools.partial(_post_body, use_ln=use_ln),
        grid=(_GRID,),
        in_specs=[
            pl.BlockSpec((NC, _RB, D), lambda i: (0, i, 0)),
            pl.BlockSpec((_RB, D), lambda i: (i, 0)),
            pl.BlockSpec((1, D), lambda i: (0, 0)),
            pl.BlockSpec((1, D), lambda i: (0, 0)),
            pl.BlockSpec((1, D), lambda i: (0, 0)),
            pl.BlockSpec((D, D), lambda i: (0, 0)),
        ],
        out_specs=pl.BlockSpec((_RB, D), lambda i: (i, 0)),
        out_shape=jax.ShapeDtypeStruct((N, D), _f32),
    )(part, dinv_b, bias, gamma, beta, w_next)


def _final_body(p_ref, dinv_ref, bias_ref, o_ref):
    o_ref[...] = dinv_ref[...] * (p_ref[0] + p_ref[1]) + bias_ref[...]


def _tc_final(part, dinv_b, bias):
    return pl.pallas_call(
        _final_body,
        grid=(_GRID,),
        in_specs=[
            pl.BlockSpec((NC, _RB, D), lambda i: (0, i, 0)),
            pl.BlockSpec((_RB, D), lambda i: (i, 0)),
            pl.BlockSpec((1, D), lambda i: (0, 0)),
        ],
        out_specs=pl.BlockSpec((_RB, D), lambda i: (i, 0)),
        out_shape=jax.ShapeDtypeStruct((N, D), _f32),
    )(part, dinv_b, bias)


def kernel(x, hyperedge_index, W1, b1, W2, b2, W3, b3, W4, b4, W5, b5,
           gamma, beta):
    nidx = hyperedge_index[0]
    eidx = hyperedge_index[1]
    zeros_big = jnp.zeros((NPAD, D), _f32)
    ones_rows = jnp.ones((C, D), _f32)

    dcnt, bcnt = _sc_counts(nidx, eidx, ones_rows, zeros_big)
    dinv_b, binv_b = _tc_inv(dcnt, bcnt)

    Ws = [W1, W2, W3, W4, W5]
    bs = [b.reshape(1, D) for b in (b1, b2, b3, b4, b5)]
    gamma2 = gamma.reshape(1, D)
    beta2 = beta.reshape(1, D)

    xw = _tc_mm(x, W1)
    for i in range(5):
        pA = _sc_stage(xw, nidx, eidx, zeros_big)
        ef = _tc_combine(pA, binv_b)
        pB = _sc_stage(ef, eidx, nidx, zeros_big)
        if i < 4:
            xw = _tc_post(pB, dinv_b, bs[i], gamma2, beta2, Ws[i + 1],
                          use_ln=(i == 0))
        else:
            z = _tc_final(pB, dinv_b, bs[4])
    return z
